# baseline (speedup 1.0000x reference)
import jax
import jax.numpy as jnp
from jax.experimental import pallas as pl

_K = 1024
_C = 64


def _match_body(x_ref, bank_ref, recon_ref, idx_ref):
    xb = x_ref[0]
    bank = bank_ref[...]
    hw = xb.shape[1]
    nsq = jnp.sum(bank * bank, axis=1, keepdims=True)
    pn = bank / jnp.maximum(jnp.sqrt(nsq), 1e-12)
    xsq = jnp.sum(xb * xb, axis=0, keepdims=True)
    xn = xb / jnp.maximum(jnp.sqrt(xsq), 1e-12)
    s = jnp.dot(pn, xn, preferred_element_type=jnp.float32)
    m = jnp.max(s, axis=0, keepdims=True)
    iota = jax.lax.broadcasted_iota(jnp.int32, (_K, hw), 0)
    idx = jnp.min(jnp.where(s >= m, iota, _K), axis=0, keepdims=True)
    onehot = (iota == idx).astype(jnp.float32)
    recon = jax.lax.dot_general(
        bank, onehot, (((0,), (0,)), ((), ())),
        preferred_element_type=jnp.float32,
        precision=jax.lax.Precision.HIGHEST)
    recon_ref[0] = recon
    idx_ref[0] = idx


def kernel(x, prototype_bank):
    B, C, H, W = x.shape
    HW = H * W
    x3 = x.reshape(B, C, HW)
    recon3, idx3 = pl.pallas_call(
        _match_body,
        grid=(B,),
        in_specs=[
            pl.BlockSpec((1, C, HW), lambda b: (b, 0, 0)),
            pl.BlockSpec((_K, C), lambda b: (0, 0)),
        ],
        out_specs=[
            pl.BlockSpec((1, C, HW), lambda b: (b, 0, 0)),
            pl.BlockSpec((1, 1, HW), lambda b: (b, 0, 0)),
        ],
        out_shape=[
            jax.ShapeDtypeStruct((B, C, HW), jnp.float32),
            jax.ShapeDtypeStruct((B, 1, HW), jnp.int32),
        ],
    )(x3, prototype_bank)
    return recon3.reshape(B, C, H, W), idx3.reshape(B, HW)


# speedup vs baseline: 1.3170x; 1.3170x over previous
import jax
import jax.numpy as jnp
from jax.experimental import pallas as pl

_K = 1024
_C = 64


def _match_body(x_ref, bank_ref, recon_ref, idx_ref):
    xb = x_ref[0]
    bank = bank_ref[...]
    hw = xb.shape[1]
    nsq = jnp.sum(bank * bank, axis=1, keepdims=True)
    pn = bank / jnp.maximum(jnp.sqrt(nsq), 1e-12)
    xsq = jnp.sum(xb * xb, axis=0, keepdims=True)
    xn = xb / jnp.maximum(jnp.sqrt(xsq), 1e-12)
    s = jnp.dot(pn, xn, preferred_element_type=jnp.float32)
    m = jnp.max(s, axis=0, keepdims=True)
    iota = jax.lax.broadcasted_iota(jnp.int32, (_K, hw), 0)
    idx = jnp.min(jnp.where(s >= m, iota, _K), axis=0, keepdims=True)
    onehot = (iota == idx).astype(jnp.float32)
    recon = jax.lax.dot_general(
        bank, onehot, (((0,), (0,)), ((), ())),
        preferred_element_type=jnp.float32)
    recon_ref[0] = recon
    idx_ref[0] = idx


def kernel(x, prototype_bank):
    B, C, H, W = x.shape
    HW = H * W
    x3 = x.reshape(B, C, HW)
    recon3, idx3 = pl.pallas_call(
        _match_body,
        grid=(B,),
        in_specs=[
            pl.BlockSpec((1, C, HW), lambda b: (b, 0, 0)),
            pl.BlockSpec((_K, C), lambda b: (0, 0)),
        ],
        out_specs=[
            pl.BlockSpec((1, C, HW), lambda b: (b, 0, 0)),
            pl.BlockSpec((1, 1, HW), lambda b: (b, 0, 0)),
        ],
        out_shape=[
            jax.ShapeDtypeStruct((B, C, HW), jnp.float32),
            jax.ShapeDtypeStruct((B, 1, HW), jnp.int32),
        ],
    )(x3, prototype_bank)
    return recon3.reshape(B, C, H, W), idx3.reshape(B, HW)


# speedup vs baseline: 1.3912x; 1.0563x over previous
import jax
import jax.numpy as jnp
from jax.experimental import pallas as pl

_K = 1024
_C = 64


def _match_body(x_ref, bank_ref, recon_ref, idx_ref):
    B = x_ref.shape[0]
    hw = x_ref.shape[2]
    bank = bank_ref[...]
    nsq = jnp.sum(bank * bank, axis=1, keepdims=True)
    pn = bank / jnp.maximum(jnp.sqrt(nsq), 1e-12)
    iota = jax.lax.broadcasted_iota(jnp.int32, (_K, hw), 0)
    for b in range(B):
        xb = x_ref[b]
        xsq = jnp.sum(xb * xb, axis=0, keepdims=True)
        xn = xb / jnp.maximum(jnp.sqrt(xsq), 1e-12)
        s = jnp.dot(pn, xn, preferred_element_type=jnp.float32)
        m = jnp.max(s, axis=0, keepdims=True)
        idx = jnp.min(jnp.where(s >= m, iota, _K), axis=0,
                      keepdims=True)
        onehot = (iota == idx).astype(jnp.float32)
        recon = jax.lax.dot_general(
            bank, onehot, (((0,), (0,)), ((), ())),
            preferred_element_type=jnp.float32)
        recon_ref[b] = recon
        idx_ref[b] = idx


def kernel(x, prototype_bank):
    B, C, H, W = x.shape
    HW = H * W
    x3 = x.reshape(B, C, HW)
    recon3, idx3 = pl.pallas_call(
        _match_body,
        out_shape=[
            jax.ShapeDtypeStruct((B, C, HW), jnp.float32),
            jax.ShapeDtypeStruct((B, 1, HW), jnp.int32),
        ],
    )(x3, prototype_bank)
    return recon3.reshape(B, C, H, W), idx3.reshape(B, HW)


# speedup vs baseline: 1.5244x; 1.0958x over previous
import jax
import jax.numpy as jnp
from jax.experimental import pallas as pl

_K = 1024
_C = 64


def _match_body(x_ref, bank_ref, recon_ref, idx_ref):
    B = x_ref.shape[0]
    hw = x_ref.shape[2]
    bank = bank_ref[...]
    nsq = jnp.sum(bank * bank, axis=1, keepdims=True)
    pn = bank / jnp.maximum(jnp.sqrt(nsq), 1e-12)
    iota = jax.lax.broadcasted_iota(jnp.int32, (_K, hw), 0)
    for b in range(B):
        xb = x_ref[b]
        xsq = jnp.sum(xb * xb, axis=0, keepdims=True)
        xn = xb / jnp.maximum(jnp.sqrt(xsq), 1e-12)
        s = jnp.dot(pn, xn, preferred_element_type=jnp.float32)
        idx = jnp.argmax(s, axis=0)[None, :]
        onehot = (iota == idx).astype(jnp.float32)
        recon = jax.lax.dot_general(
            bank, onehot, (((0,), (0,)), ((), ())),
            preferred_element_type=jnp.float32)
        recon_ref[b] = recon
        idx_ref[b] = idx


def kernel(x, prototype_bank):
    B, C, H, W = x.shape
    HW = H * W
    x3 = x.reshape(B, C, HW)
    recon3, idx3 = pl.pallas_call(
        _match_body,
        out_shape=[
            jax.ShapeDtypeStruct((B, C, HW), jnp.float32),
            jax.ShapeDtypeStruct((B, 1, HW), jnp.int32),
        ],
    )(x3, prototype_bank)
    return recon3.reshape(B, C, H, W), idx3.reshape(B, HW)
